# Initial kernel scaffold; baseline (speedup 1.0000x reference)
import jax
import jax.numpy as jnp
from jax.experimental import pallas as pl


def kernel(x, edge_index, W0, b0, g0, be0, W1, b1, g1, be1):
    raise NotImplementedError("write your pallas kernel here")


# speedup vs baseline: 11.2144x; 11.2144x over previous
import functools

import jax
import jax.numpy as jnp
from jax import lax
from jax.experimental import pallas as pl
from jax.experimental.pallas import tpu as pltpu
from jax.experimental.pallas import tpu_sc as plsc

NC = 2
NS = 16
CH = 128
W16 = 16


def _sc_mesh():
    return plsc.VectorSubcoreMesh(core_axis_name="c", subcore_axis_name="s")


def _deg_partials(dst_w, T, AR, NCH, D):
    @functools.partial(
        pl.kernel,
        out_type=jax.ShapeDtypeStruct((NC, T, D), jnp.float32),
        mesh=_sc_mesh(),
        scratch_types=[
            pltpu.VMEM((NCH, CH), jnp.int32),
            pltpu.VMEM((CH, D), jnp.float32),
            pltpu.VMEM_SHARED((T, D), jnp.float32),
        ],
    )
    def deg_k(dstw_h, degp_h, idx_v, buf_v, dacc):
        cid = lax.axis_index("c")
        sid = lax.axis_index("s")
        w = cid * NS + sid
        pltpu.sync_copy(dstw_h.at[w], idx_v)
        for i in range(CH):
            for c in range(D // 16):
                buf_v[i, pl.ds(c * 16, 16)] = jnp.zeros((16,), jnp.float32)
        base = sid * AR
        for c in range(AR // CH):
            pltpu.sync_copy(buf_v, dacc.at[pl.ds(base + c * CH, CH)])
        for i in range(CH):
            for c in range(D // 16):
                buf_v[i, pl.ds(c * 16, 16)] = jnp.ones((16,), jnp.float32)
        plsc.subcore_barrier()
        for j in range(NCH):
            pltpu.sync_copy(buf_v, dacc.at[idx_v.at[j]], add=True)
        plsc.subcore_barrier()
        for c in range(AR // CH):
            pltpu.sync_copy(dacc.at[pl.ds(base + c * CH, CH)], buf_v)
            pltpu.sync_copy(buf_v, degp_h.at[cid, pl.ds(base + c * CH, CH)])

    return deg_k(dst_w)


def _agg_partials(xs, src_w, dst_w, T, AR, NCH, D):
    @functools.partial(
        pl.kernel,
        out_type=jax.ShapeDtypeStruct((NC, T, D), jnp.float32),
        mesh=_sc_mesh(),
        scratch_types=[
            pltpu.VMEM((NCH, CH), jnp.int32),
            pltpu.VMEM((NCH, CH), jnp.int32),
            pltpu.VMEM((CH, D), jnp.float32),
            pltpu.VMEM_SHARED((T, D), jnp.float32),
        ],
    )
    def agg_k(xs_h, srcw_h, dstw_h, aggp_h, sidx, didx, rows, acc):
        cid = lax.axis_index("c")
        sid = lax.axis_index("s")
        w = cid * NS + sid
        pltpu.sync_copy(srcw_h.at[w], sidx)
        pltpu.sync_copy(dstw_h.at[w], didx)
        for i in range(CH):
            for c in range(D // 16):
                rows[i, pl.ds(c * 16, 16)] = jnp.zeros((16,), jnp.float32)
        base = sid * AR
        for c in range(AR // CH):
            pltpu.sync_copy(rows, acc.at[pl.ds(base + c * CH, CH)])
        plsc.subcore_barrier()
        for j in range(NCH):
            pltpu.sync_copy(xs_h.at[sidx.at[j]], rows)
            pltpu.sync_copy(rows, acc.at[didx.at[j]], add=True)
        plsc.subcore_barrier()
        for c in range(AR // CH):
            pltpu.sync_copy(acc.at[pl.ds(base + c * CH, CH)], rows)
            pltpu.sync_copy(rows, aggp_h.at[cid, pl.ds(base + c * CH, CH)])

    return agg_k(xs, src_w, dst_w)


def _tc_prep(x, W0, degp, N, T, D):
    def body(x_ref, w_ref, dp_ref, xs_ref, dinv_ref):
        deg = dp_ref[0, :, 0:1] + dp_ref[1, :, 0:1] + 1.0
        dinv = lax.rsqrt(deg)
        dinv_ref[...] = dinv
        xw = jnp.dot(x_ref[...], w_ref[...], preferred_element_type=jnp.float32)
        xs_ref[...] = xw * dinv[:N]

    return pl.pallas_call(
        body,
        out_shape=(
            jax.ShapeDtypeStruct((N, D), jnp.float32),
            jax.ShapeDtypeStruct((T, 1), jnp.float32),
        ),
    )(x, W0, degp)


def _bn(h, g, be):
    m = jnp.mean(h, axis=0, keepdims=True)
    v = jnp.mean((h - m) * (h - m), axis=0, keepdims=True)
    return (h - m) * lax.rsqrt(v + 1e-5) * g + be


def _tc_mid(aggp, xs0, dinv, b0, g0, be0, W1, N, D):
    def body(ap_ref, xs_ref, dv_ref, b_ref, g_ref, be_ref, w_ref, o_ref):
        agg = ap_ref[0, :N, :] + ap_ref[1, :N, :]
        dinv = dv_ref[:N]
        h = dinv * (agg + xs_ref[...]) + b_ref[...]
        y = jnp.maximum(_bn(h, g_ref[...], be_ref[...]), 0.0)
        o_ref[...] = (
            jnp.dot(y, w_ref[...], preferred_element_type=jnp.float32) * dinv
        )

    return pl.pallas_call(
        body,
        out_shape=jax.ShapeDtypeStruct((N, D), jnp.float32),
    )(aggp, xs0, dinv, b0, g0, be0, W1)


def _tc_final(aggp, xs1, dinv, b1, g1, be1, x, N, D):
    def body(ap_ref, xs_ref, dv_ref, b_ref, g_ref, be_ref, x_ref, o_ref):
        agg = ap_ref[0, :N, :] + ap_ref[1, :N, :]
        dinv = dv_ref[:N]
        h = dinv * (agg + xs_ref[...]) + b_ref[...]
        y = _bn(h, g_ref[...], be_ref[...])
        o_ref[...] = jnp.maximum(y + x_ref[...], 0.0)

    return pl.pallas_call(
        body,
        out_shape=jax.ShapeDtypeStruct((N, D), jnp.float32),
    )(aggp, xs1, dinv, b1, g1, be1, x)


def kernel(x, edge_index, W0, b0, g0, be0, W1, b1, g1, be1):
    N, D = x.shape
    E = edge_index.shape[1]
    NW = NC * NS
    NCH = -(-E // (NW * CH))
    EP = NW * NCH * CH
    AR = (-(-(N + 1) // (NS * CH))) * CH
    T = NS * AR

    src = edge_index[0]
    dst = edge_index[1]
    src_w = jnp.concatenate(
        [src, jnp.zeros((EP - E,), jnp.int32)]).reshape(NW, NCH, CH)
    dst_w = jnp.concatenate(
        [dst, jnp.full((EP - E,), N, jnp.int32)]).reshape(NW, NCH, CH)

    degp = _deg_partials(dst_w, T, AR, NCH, D)
    xs0, dinv = _tc_prep(x, W0, degp, N, T, D)
    aggp0 = _agg_partials(xs0, src_w, dst_w, T, AR, NCH, D)
    xs1 = _tc_mid(aggp0, xs0, dinv, b0.reshape(1, D), g0.reshape(1, D),
                  be0.reshape(1, D), W1, N, D)
    aggp1 = _agg_partials(xs1, src_w, dst_w, T, AR, NCH, D)
    return _tc_final(aggp1, xs1, dinv, b1.reshape(1, D), g1.reshape(1, D),
                     be1.reshape(1, D), x, N, D)
